# Initial kernel scaffold; baseline (speedup 1.0000x reference)
import jax
import jax.numpy as jnp
from jax.experimental import pallas as pl


def kernel(a, s, pe, ne, kp):
    raise NotImplementedError("write your pallas kernel here")


# speedup vs baseline: 1.7938x; 1.7938x over previous
import functools

import jax
import jax.numpy as jnp
from jax.experimental import pallas as pl
from jax.experimental.pallas import tpu as pltpu

N = 8000
TSTEPS = 199
REG_BLOCK_ROWS = 200


def _scan_body(a_ref, s_ref, pe_row, ne_row, kp_ref, out_ref, k_ref, st_ref):
    i = pl.program_id(0)

    @pl.when(i == 0)
    def _init():
        k_ref[...] = kp_ref[...]
        st_ref[0] = 0.0
        st_ref[1] = 1.0

    ai = a_ref[i]
    si = s_ref[i]
    active = st_ref[1] > 0.5
    cond = jnp.logical_and(active, si >= 0.0)

    k = k_ref[...]
    col = jax.lax.broadcasted_iota(jnp.int32, (1, N), 1)
    kval = jnp.sum(jnp.where(col == ai, k, 0.0))
    p = jnp.clip(kval, 0.01, 0.99)
    loss = st_ref[0]
    l_new = loss - (si * jnp.log(p) + (1.0 - si) * jnp.log(1.0 - p))
    k_new = jnp.clip(
        k + si * pe_row[...].reshape(1, N) + (1.0 - si) * ne_row[...].reshape(1, N),
        -30.0,
        30.0,
    )
    k_ref[...] = jnp.where(cond, k_new, k)
    st_ref[0] = jnp.where(cond, l_new, loss)
    st_ref[1] = jnp.where(cond, 1.0, 0.0)

    @pl.when(i == pl.num_programs(0) - 1)
    def _fini():
        out_ref[0] = st_ref[0]


def _reg_body(pe_blk, ne_blk, out_ref, acc_ref):
    i = pl.program_id(0)

    @pl.when(i == 0)
    def _init():
        acc_ref[0] = 0.0

    x = pe_blk[...]
    y = ne_blk[...]
    acc_ref[0] += jnp.sum(jnp.log(jnp.abs(x) + 1.0)) + jnp.sum(
        jnp.log(jnp.abs(y) + 1.0)
    )

    @pl.when(i == pl.num_programs(0) - 1)
    def _fini():
        out_ref[0] = acc_ref[0]


def _loss_call(a, s, pe, ne, kp, interpret=False):
    grid_spec = pltpu.PrefetchScalarGridSpec(
        num_scalar_prefetch=2,
        grid=(TSTEPS,),
        in_specs=[
            pl.BlockSpec((1, 1, N), lambda i, a_ref, s_ref: (a_ref[i], 0, 0)),
            pl.BlockSpec((1, 1, N), lambda i, a_ref, s_ref: (a_ref[i], 0, 0)),
            pl.BlockSpec((1, N), lambda i, a_ref, s_ref: (0, 0)),
        ],
        out_specs=pl.BlockSpec(memory_space=pltpu.SMEM),
        scratch_shapes=[
            pltpu.VMEM((1, N), jnp.float32),
            pltpu.SMEM((2,), jnp.float32),
        ],
    )
    return pl.pallas_call(
        _scan_body,
        grid_spec=grid_spec,
        out_shape=jax.ShapeDtypeStruct((1,), jnp.float32),
        interpret=interpret,
    )(a, s, pe.reshape(N, 1, N), ne.reshape(N, 1, N), kp.reshape(1, N))[0]


def _reg_call(pe, ne, interpret=False):
    nblk = N // REG_BLOCK_ROWS
    return pl.pallas_call(
        _reg_body,
        grid=(nblk,),
        in_specs=[
            pl.BlockSpec((REG_BLOCK_ROWS, N), lambda i: (i, 0)),
            pl.BlockSpec((REG_BLOCK_ROWS, N), lambda i: (i, 0)),
        ],
        out_specs=pl.BlockSpec(memory_space=pltpu.SMEM),
        out_shape=jax.ShapeDtypeStruct((1,), jnp.float32),
        scratch_shapes=[pltpu.SMEM((1,), jnp.float32)],
        interpret=interpret,
    )(pe, ne)[0]


@functools.partial(jax.jit, static_argnames=("interpret",))
def _kernel_impl(a, s, pe, ne, kp, interpret=False):
    a32 = a[:TSTEPS].astype(jnp.int32)
    s32 = s[:TSTEPS].astype(jnp.float32)
    loss = _loss_call(a32, s32, pe, ne, kp, interpret=interpret)
    reg = _reg_call(pe, ne, interpret=interpret)
    return loss + 0.5 * reg


def kernel(a, s, pe, ne, kp):
    return _kernel_impl(a, s, pe, ne, kp)
